# baseline (speedup 1.0000x reference)
import functools

import jax
import jax.numpy as jnp
from jax import lax
from jax.experimental import pallas as pl
from jax.experimental.pallas import tpu as pltpu
from jax.experimental.pallas import tpu_sc as plsc

N = 10000
E = 320000
D = 128
HD = D // 2

NC = 2
NS = 16
NW = NC * NS

EPT = E // NW
K = 80
NCH = EPT // K
RPT = 624
TAIL = N - NS * RPT
ZR = 208
LANES = 16


GE = 16


NB = 4


def _wscatter_kernel(y0_hbm, y1_hbm, src_hbm, dst_hbm, w_hbm,
                     out0_hbm, out1_hbm,
                     src_v, dst_v, w_v, rows0, rows1, rows2, rows3,
                     rowso0, rowso1, rowso2, rowso3, zbuf, acc,
                     g0, g1, g2, g3, s0, s1, s2, s3):
    c = lax.axis_index("c")
    s = lax.axis_index("s")
    wid = s * NC + c

    pltpu.sync_copy(src_hbm.at[wid], src_v)
    pltpu.sync_copy(dst_hbm.at[wid], dst_v)
    pltpu.sync_copy(w_hbm.at[wid], w_v)

    zv = jnp.zeros((LANES,), jnp.float32)

    def zrow(r, carry):
        for f in range(HD // LANES):
            zbuf[r, pl.ds(f * LANES, LANES)] = zv
        return carry

    lax.fori_loop(0, ZR, zrow, 0)

    def scale(rows, rowso, ch):
        for g in range(K // GE):
            w16 = w_v[pl.ds(ch * K + g * GE, GE)]
            for u in range(GE):
                idx = jnp.full((LANES, 1), u, jnp.int32)
                wv = lax.gather(
                    w16, idx,
                    lax.GatherDimensionNumbers(
                        offset_dims=(), collapsed_slice_dims=(0,),
                        start_index_map=(0,)),
                    (1,),
                    mode=lax.GatherScatterMode.PROMISE_IN_BOUNDS)
                r = g * GE + u
                for f in range(HD // LANES):
                    sl = pl.ds(f * LANES, LANES)
                    rowso[r, sl] = rows[r, sl] * wv

    for y_hbm, out_hbm in ((y0_hbm, out0_hbm), (y1_hbm, out1_hbm)):
        def zcp(k, carry):
            pltpu.sync_copy(zbuf, acc.at[pl.ds(s * RPT + k * ZR, ZR)])
            return carry

        lax.fori_loop(0, RPT // ZR, zcp, 0)

        @pl.when(s == NS - 1)
        def _():
            pltpu.sync_copy(zbuf.at[pl.ds(0, TAIL)], acc.at[pl.ds(NS * RPT, TAIL)])

        plsc.subcore_barrier()

        rows = (rows0, rows1, rows2, rows3)
        rowso = (rowso0, rowso1, rowso2, rowso3)
        gsem = (g0, g1, g2, g3)
        ssem = (s0, s1, s2, s3)
        for b in range(NB):
            pltpu.async_copy(y_hbm.at[src_v.at[b]], rows[b], gsem[b])

        def ring(jj, carry):
            base_ch = NB * jj
            for b in range(NB):
                ch = base_ch + b
                pltpu.make_async_copy(y_hbm.at[src_v.at[ch]], rows[b], gsem[b]).wait()

                @pl.when(jj > 0)
                def _():
                    pltpu.make_async_copy(rowso[b], acc.at[dst_v.at[ch]], ssem[b]).wait()

                scale(rows[b], rowso[b], ch)
                pltpu.async_copy(rowso[b], acc.at[dst_v.at[ch]], ssem[b], add=True)
                fetch = ch + NB
                if b == 0:
                    pltpu.async_copy(y_hbm.at[src_v.at[fetch]], rows[b], gsem[b])
                else:
                    @pl.when(fetch < NCH)
                    def _():
                        pltpu.async_copy(y_hbm.at[src_v.at[fetch]], rows[b], gsem[b])
            return carry

        lax.fori_loop(0, (NCH - 1) // NB, ring, 0)

        last = NCH - 1
        pltpu.make_async_copy(y_hbm.at[src_v.at[last]], rows[0], gsem[0]).wait()
        pltpu.make_async_copy(rowso[0], acc.at[dst_v.at[last]], ssem[0]).wait()
        scale(rows[0], rowso[0], last)
        pltpu.async_copy(rowso[0], acc.at[dst_v.at[last]], ssem[0], add=True)
        pltpu.make_async_copy(rowso[0], acc.at[dst_v.at[last]], ssem[0]).wait()
        for b in range(1, NB):
            pltpu.make_async_copy(rowso[b], acc.at[dst_v.at[last - NB + b]], ssem[b]).wait()

        plsc.subcore_barrier()

        sl = pl.ds(s * RPT, RPT)
        pltpu.sync_copy(acc.at[sl], out_hbm.at[c, sl, pl.ds(0, HD)])

        @pl.when(s == NS - 1)
        def _():
            tl = pl.ds(NS * RPT, TAIL)
            pltpu.sync_copy(acc.at[tl], out_hbm.at[c, tl, pl.ds(0, HD)])


def _wscatter(y0, y1, src, dst, w):
    mesh = plsc.VectorSubcoreMesh(core_axis_name="c", subcore_axis_name="s",
                                  num_cores=NC, num_subcores=NS)
    fn = pl.kernel(
        _wscatter_kernel,
        out_type=(jax.ShapeDtypeStruct((NC, N, D), jnp.float32),
                  jax.ShapeDtypeStruct((NC, N, D), jnp.float32)),
        mesh=mesh,
        scratch_types=[
            pltpu.VMEM((NCH, K), jnp.int32),
            pltpu.VMEM((NCH, K), jnp.int32),
            pltpu.VMEM((EPT,), jnp.float32),
        ] + [pltpu.VMEM((K, HD), jnp.float32)] * (2 * NB) + [
            pltpu.VMEM((ZR, HD), jnp.float32),
            pltpu.VMEM_SHARED((N, HD), jnp.float32),
        ] + [pltpu.SemaphoreType.DMA] * (2 * NB),
        compiler_params=pltpu.CompilerParams(needs_layout_passes=False,
                                             use_tc_tiling_on_sc=False),
    )
    return fn(y0, y1, src, dst, w)


BM = 2000


def _mm_split_body(x_ref, w_ref, o0_ref, o1_ref):
    res = jnp.dot(x_ref[...], w_ref[...], preferred_element_type=jnp.float32)
    r3 = res.reshape(BM // 2, 2, D)
    o0_ref[...] = jnp.concatenate([r3[:, 0, :HD], r3[:, 1, :HD]], axis=-1)
    o1_ref[...] = jnp.concatenate([r3[:, 0, HD:], r3[:, 1, HD:]], axis=-1)


def _mm_split(x, w):
    return pl.pallas_call(
        _mm_split_body,
        grid=(N // BM,),
        in_specs=[
            pl.BlockSpec((BM, D), lambda i: (i, 0)),
            pl.BlockSpec((D, D), lambda i: (0, 0)),
        ],
        out_specs=(pl.BlockSpec((BM // 2, D), lambda i: (i, 0)),
                   pl.BlockSpec((BM // 2, D), lambda i: (i, 0))),
        out_shape=(jax.ShapeDtypeStruct((N // 2, D), jnp.float32),
                   jax.ShapeDtypeStruct((N // 2, D), jnp.float32)),
    )(x, w)


def _assemble(p0_ref, p1_ref, b_ref):
    left = p0_ref[0, :, :HD] + p0_ref[1, :, :HD]
    right = p1_ref[0, :, :HD] + p1_ref[1, :, :HD]
    return jnp.concatenate([left, right], axis=-1) + b_ref[...]


def _relu_mm_split_body(p0_ref, p1_ref, b_ref, w_ref, o0_ref, o1_ref):
    h = jnp.maximum(_assemble(p0_ref, p1_ref, b_ref), 0.0)
    res = jnp.dot(h, w_ref[...], preferred_element_type=jnp.float32)
    r3 = res.reshape(BM // 2, 2, D)
    o0_ref[...] = jnp.concatenate([r3[:, 0, :HD], r3[:, 1, :HD]], axis=-1)
    o1_ref[...] = jnp.concatenate([r3[:, 0, HD:], r3[:, 1, HD:]], axis=-1)


def _relu_mm_split(p0, p1, b, w):
    part_spec = pl.BlockSpec((NC, BM, D), lambda i: (0, i, 0))
    return pl.pallas_call(
        _relu_mm_split_body,
        grid=(N // BM,),
        in_specs=[
            part_spec,
            part_spec,
            pl.BlockSpec((1, D), lambda i: (0, 0)),
            pl.BlockSpec((D, D), lambda i: (0, 0)),
        ],
        out_specs=(pl.BlockSpec((BM // 2, D), lambda i: (i, 0)),
                   pl.BlockSpec((BM // 2, D), lambda i: (i, 0))),
        out_shape=(jax.ShapeDtypeStruct((N // 2, D), jnp.float32),
                   jax.ShapeDtypeStruct((N // 2, D), jnp.float32)),
    )(p0, p1, b.reshape(1, D), w)


def _combine_body(p0_ref, p1_ref, b_ref, o_ref):
    o_ref[...] = _assemble(p0_ref, p1_ref, b_ref)


def _combine(p0, p1, b):
    part_spec = pl.BlockSpec((NC, BM, D), lambda i: (0, i, 0))
    return pl.pallas_call(
        _combine_body,
        grid=(N // BM,),
        in_specs=[
            part_spec,
            part_spec,
            pl.BlockSpec((1, D), lambda i: (0, 0)),
        ],
        out_specs=pl.BlockSpec((BM, D), lambda i: (i, 0)),
        out_shape=jax.ShapeDtypeStruct((N, D), jnp.float32),
    )(p0, p1, b.reshape(1, D))


@jax.jit
def kernel(x, edge_index, edge_weight, node_type, W1, b1, W2, b2):
    del node_type
    ei = edge_index.astype(jnp.int32)
    src = ei[0].reshape(NW, NCH, K)
    dst = ei[1].reshape(NW, NCH, K)
    w = edge_weight.astype(jnp.float32).reshape(NW, EPT)

    y10, y11 = _mm_split(x.astype(jnp.float32), W1)
    a10, a11 = _wscatter(y10.reshape(N, HD), y11.reshape(N, HD), src, dst, w)
    y20, y21 = _relu_mm_split(a10, a11, b1, W2)
    a20, a21 = _wscatter(y20.reshape(N, HD), y21.reshape(N, HD), src, dst, w)
    return _combine(a20, a21, b2)


# speedup vs baseline: 1.1424x; 1.1424x over previous
import functools

import jax
import jax.numpy as jnp
from jax import lax
from jax.experimental import pallas as pl
from jax.experimental.pallas import tpu as pltpu
from jax.experimental.pallas import tpu_sc as plsc

N = 10000
E = 320000
D = 128
HD = D // 2

NC = 2
NS = 16
NW = NC * NS

EPT = E // NW
K = 80
NCH = EPT // K
RPT = 624
TAIL = N - NS * RPT
ZR = 208
LANES = 16


GE = 16


NB = 4


def _wscatter_kernel(y0_hbm, y1_hbm, src_hbm, dst_hbm, w_hbm, out_hbm,
                     src_v, dst_v, w_v, rows0, rows1, rows2, rows3,
                     rowso0, rowso1, rowso2, rowso3, zbuf, acc,
                     g0, g1, g2, g3, s0, s1, s2, s3):
    c = lax.axis_index("c")
    s = lax.axis_index("s")
    wid = s * NC + c

    pltpu.sync_copy(src_hbm.at[wid], src_v)
    pltpu.sync_copy(dst_hbm.at[wid], dst_v)
    pltpu.sync_copy(w_hbm.at[wid], w_v)

    zv = jnp.zeros((LANES,), jnp.float32)

    def zrow(r, carry):
        for f in range(HD // LANES):
            zbuf[r, pl.ds(f * LANES, LANES)] = zv
        return carry

    lax.fori_loop(0, ZR, zrow, 0)

    def scale(rows, rowso, ch):
        def grp(g, carry):
            w16 = w_v[pl.ds(ch * K + g * GE, GE)]
            for u in range(GE):
                idx = jnp.full((LANES, 1), u, jnp.int32)
                wv = lax.gather(
                    w16, idx,
                    lax.GatherDimensionNumbers(
                        offset_dims=(), collapsed_slice_dims=(0,),
                        start_index_map=(0,)),
                    (1,),
                    mode=lax.GatherScatterMode.PROMISE_IN_BOUNDS)
                r = g * GE + u
                for f in range(HD // LANES):
                    sl = pl.ds(f * LANES, LANES)
                    rowso[r, sl] = rows[r, sl] * wv
            return carry

        lax.fori_loop(0, K // GE, grp, 0)

    for col, y_hbm in ((0, y0_hbm), (HD, y1_hbm)):
        def zcp(k, carry):
            pltpu.sync_copy(zbuf, acc.at[pl.ds(s * RPT + k * ZR, ZR)])
            return carry

        lax.fori_loop(0, RPT // ZR, zcp, 0)

        @pl.when(s == NS - 1)
        def _():
            pltpu.sync_copy(zbuf.at[pl.ds(0, TAIL)], acc.at[pl.ds(NS * RPT, TAIL)])

        plsc.subcore_barrier()

        rows = (rows0, rows1, rows2, rows3)
        rowso = (rowso0, rowso1, rowso2, rowso3)
        gsem = (g0, g1, g2, g3)
        ssem = (s0, s1, s2, s3)
        for b in range(NB):
            pltpu.async_copy(y_hbm.at[src_v.at[b]], rows[b], gsem[b])

        def ring(jj, carry):
            base_ch = NB * jj
            for b in range(NB):
                ch = base_ch + b
                pltpu.make_async_copy(y_hbm.at[src_v.at[ch]], rows[b], gsem[b]).wait()

                @pl.when(jj > 0)
                def _():
                    pltpu.make_async_copy(rowso[b], acc.at[dst_v.at[ch]], ssem[b]).wait()

                scale(rows[b], rowso[b], ch)
                pltpu.async_copy(rowso[b], acc.at[dst_v.at[ch]], ssem[b], add=True)
                fetch = ch + NB
                if b == 0:
                    pltpu.async_copy(y_hbm.at[src_v.at[fetch]], rows[b], gsem[b])
                else:
                    @pl.when(fetch < NCH)
                    def _():
                        pltpu.async_copy(y_hbm.at[src_v.at[fetch]], rows[b], gsem[b])
            return carry

        lax.fori_loop(0, (NCH - 1) // NB, ring, 0)

        last = NCH - 1
        pltpu.make_async_copy(y_hbm.at[src_v.at[last]], rows[0], gsem[0]).wait()
        pltpu.make_async_copy(rowso[0], acc.at[dst_v.at[last]], ssem[0]).wait()
        scale(rows[0], rowso[0], last)
        pltpu.async_copy(rowso[0], acc.at[dst_v.at[last]], ssem[0], add=True)
        pltpu.make_async_copy(rowso[0], acc.at[dst_v.at[last]], ssem[0]).wait()
        for b in range(1, NB):
            pltpu.make_async_copy(rowso[b], acc.at[dst_v.at[last - NB + b]], ssem[b]).wait()

        plsc.subcore_barrier()

        sl = pl.ds(s * RPT, RPT)
        pltpu.sync_copy(acc.at[sl], out_hbm.at[c, sl, pl.ds(col, HD)])

        @pl.when(s == NS - 1)
        def _():
            tl = pl.ds(NS * RPT, TAIL)
            pltpu.sync_copy(acc.at[tl], out_hbm.at[c, tl, pl.ds(col, HD)])


def _wscatter(y0, y1, src, dst, w):
    mesh = plsc.VectorSubcoreMesh(core_axis_name="c", subcore_axis_name="s",
                                  num_cores=NC, num_subcores=NS)
    fn = pl.kernel(
        _wscatter_kernel,
        out_type=jax.ShapeDtypeStruct((NC, N, D), jnp.float32),
        mesh=mesh,
        scratch_types=[
            pltpu.VMEM((NCH, K), jnp.int32),
            pltpu.VMEM((NCH, K), jnp.int32),
            pltpu.VMEM((EPT,), jnp.float32),
        ] + [pltpu.VMEM((K, HD), jnp.float32)] * (2 * NB) + [
            pltpu.VMEM((ZR, HD), jnp.float32),
            pltpu.VMEM_SHARED((N, HD), jnp.float32),
        ] + [pltpu.SemaphoreType.DMA] * (2 * NB),
        compiler_params=pltpu.CompilerParams(needs_layout_passes=False,
                                             use_tc_tiling_on_sc=False),
    )
    return fn(y0, y1, src, dst, w)


BM = 2000


def _mm_split_body(x_ref, w_ref, o0_ref, o1_ref):
    res = jnp.dot(x_ref[...], w_ref[...], preferred_element_type=jnp.float32)
    r3 = res.reshape(BM // 2, 2, D)
    o0_ref[...] = jnp.concatenate([r3[:, 0, :HD], r3[:, 1, :HD]], axis=-1)
    o1_ref[...] = jnp.concatenate([r3[:, 0, HD:], r3[:, 1, HD:]], axis=-1)


def _mm_split(x, w):
    return pl.pallas_call(
        _mm_split_body,
        grid=(N // BM,),
        in_specs=[
            pl.BlockSpec((BM, D), lambda i: (i, 0)),
            pl.BlockSpec((D, D), lambda i: (0, 0)),
        ],
        out_specs=(pl.BlockSpec((BM // 2, D), lambda i: (i, 0)),
                   pl.BlockSpec((BM // 2, D), lambda i: (i, 0))),
        out_shape=(jax.ShapeDtypeStruct((N // 2, D), jnp.float32),
                   jax.ShapeDtypeStruct((N // 2, D), jnp.float32)),
    )(x, w)


def _assemble(p_ref, b_ref):
    return p_ref[0] + p_ref[1] + b_ref[...]


def _relu_mm_split_body(p_ref, b_ref, w_ref, o0_ref, o1_ref):
    h = jnp.maximum(_assemble(p_ref, b_ref), 0.0)
    res = jnp.dot(h, w_ref[...], preferred_element_type=jnp.float32)
    r3 = res.reshape(BM // 2, 2, D)
    o0_ref[...] = jnp.concatenate([r3[:, 0, :HD], r3[:, 1, :HD]], axis=-1)
    o1_ref[...] = jnp.concatenate([r3[:, 0, HD:], r3[:, 1, HD:]], axis=-1)


def _relu_mm_split(pp, b, w):
    part_spec = pl.BlockSpec((NC, BM, D), lambda i: (0, i, 0))
    return pl.pallas_call(
        _relu_mm_split_body,
        grid=(N // BM,),
        in_specs=[
            part_spec,
            pl.BlockSpec((1, D), lambda i: (0, 0)),
            pl.BlockSpec((D, D), lambda i: (0, 0)),
        ],
        out_specs=(pl.BlockSpec((BM // 2, D), lambda i: (i, 0)),
                   pl.BlockSpec((BM // 2, D), lambda i: (i, 0))),
        out_shape=(jax.ShapeDtypeStruct((N // 2, D), jnp.float32),
                   jax.ShapeDtypeStruct((N // 2, D), jnp.float32)),
    )(pp, b.reshape(1, D), w)


def _combine_body(p_ref, b_ref, o_ref):
    o_ref[...] = _assemble(p_ref, b_ref)


def _combine(pp, b):
    part_spec = pl.BlockSpec((NC, BM, D), lambda i: (0, i, 0))
    return pl.pallas_call(
        _combine_body,
        grid=(N // BM,),
        in_specs=[
            part_spec,
            pl.BlockSpec((1, D), lambda i: (0, 0)),
        ],
        out_specs=pl.BlockSpec((BM, D), lambda i: (i, 0)),
        out_shape=jax.ShapeDtypeStruct((N, D), jnp.float32),
    )(pp, b.reshape(1, D))


@jax.jit
def kernel(x, edge_index, edge_weight, node_type, W1, b1, W2, b2):
    del node_type
    ei = edge_index.astype(jnp.int32)
    src = ei[0].reshape(NW, NCH, K)
    dst = ei[1].reshape(NW, NCH, K)
    w = edge_weight.astype(jnp.float32).reshape(NW, EPT)

    y10, y11 = _mm_split(x.astype(jnp.float32), W1)
    a1 = _wscatter(y10.reshape(N, HD), y11.reshape(N, HD), src, dst, w)
    y20, y21 = _relu_mm_split(a1, b1, W2)
    a2 = _wscatter(y20.reshape(N, HD), y21.reshape(N, HD), src, dst, w)
    return _combine(a2, b2)
